# baseline (speedup 1.0000x reference)
import functools

import jax
import jax.numpy as jnp
from jax import lax
from jax.experimental import pallas as pl
from jax.experimental.pallas import tpu as pltpu
from jax.experimental.pallas import tpu_sc as plsc

SEQ = 200
BATCH = 16384
VOCAB = 100000
LANES = 16
NC = 2
NS = 16
NW = NC * NS
CPW = BATCH // NW
W = 256
NBLOCK = CPW // W
NG = W // LANES
RS = 8
NRC = SEQ // RS
NCHUNK = NBLOCK * NRC
NIB = 4
NOB = 2
PVOCAB = VOCAB // 2


def _sc_body(seq_hbm, delta_hbm, out_hbm,
             table_sp, table_v, idx0, idx1, idx2, idx3, outb0, outb1,
             in_sem0, in_sem1, in_sem2, in_sem3, out_sem0, out_sem1):
    sid = lax.axis_index("s")
    wid = sid * NC + lax.axis_index("c")
    base = wid * CPW
    idx_refs = (idx0, idx1, idx2, idx3)
    in_sems = (in_sem0, in_sem1, in_sem2, in_sem3)
    out_refs = (outb0, outb1)
    out_sems = (out_sem0, out_sem1)

    def hbm_slice(c):
        blk = c // NRC
        rc = c - blk * NRC
        r0 = rc * RS
        c0 = base + blk * W
        return pl.ds(r0, RS), pl.ds(c0, W)

    def in_copy(c, b):
        rs, cs = hbm_slice(c)
        return pltpu.make_async_copy(seq_hbm.at[rs, cs], idx_refs[b], in_sems[b])

    def out_copy(c, b):
        rs, cs = hbm_slice(c)
        return pltpu.make_async_copy(out_refs[b], out_hbm.at[rs, cs], out_sems[b])

    for b in range(NIB):
        in_copy(b, b).start()

    @pl.when(sid == 0)
    def _():
        pltpu.sync_copy(delta_hbm, table_sp)

    plsc.subcore_barrier()
    pltpu.sync_copy(table_sp, table_v)

    def chunk_step(c, ib, ob, acc):
        in_copy(c, ib).wait()

        @pl.when(c >= NOB)
        def _():
            out_copy(c - NOB, ob).wait()

        rc = c - (c // NRC) * NRC
        reset = rc == 0
        zero = jnp.zeros((LANES,), jnp.float32)
        acc = tuple(jnp.where(reset, zero, a) for a in acc)

        idx_ref = idx_refs[ib]
        out_ref = out_refs[ob]

        def row(l, acc):
            idxs = [idx_ref[l, pl.ds(g * LANES, LANES)] for g in range(NG)]
            words = [plsc.load_gather(table_v, [lax.shift_right_logical(idxs[g], 1)])
                     for g in range(NG)]
            shs = [lax.shift_left(jnp.bitwise_and(idxs[g], 1), 4) for g in range(NG)]
            vals = [plsc.bitcast(
                        lax.shift_left(lax.shift_right_logical(words[g], shs[g]), 16),
                        jnp.float32)
                    for g in range(NG)]
            new = tuple(acc[g] + vals[g] for g in range(NG))
            for g in range(NG):
                out_ref[l, pl.ds(g * LANES, LANES)] = new[g]
            return new

        acc = lax.fori_loop(0, RS, row, acc)
        out_copy(c, ob).start()

        @pl.when(c + NIB < NCHUNK)
        def _():
            in_copy(c + NIB, ib).start()

        return acc

    def quad(i, acc):
        for b in range(NIB):
            acc = chunk_step(NIB * i + b, b, b % NOB, acc)
        return acc

    zero = tuple(jnp.zeros((LANES,), jnp.float32) for _ in range(NG))
    acc = lax.fori_loop(0, NCHUNK // NIB, quad, zero)

    for c in range(NIB * (NCHUNK // NIB), NCHUNK):
        acc = chunk_step(c, c % NIB, c % NOB, acc)

    out_copy(NCHUNK - 2, (NCHUNK - 2) % NOB).wait()
    out_copy(NCHUNK - 1, (NCHUNK - 1) % NOB).wait()


def kernel(input_seq, delta):
    packed = jax.lax.bitcast_convert_type(
        delta.astype(jnp.bfloat16).reshape(PVOCAB, 2), jnp.int32)
    mesh = plsc.VectorSubcoreMesh(core_axis_name="c", subcore_axis_name="s")
    run = pl.kernel(
        _sc_body,
        mesh=mesh,
        compiler_params=pltpu.CompilerParams(needs_layout_passes=False),
        out_type=jax.ShapeDtypeStruct((SEQ, BATCH), jnp.float32),
        scratch_types=[
            pltpu.VMEM_SHARED((PVOCAB,), jnp.int32),
            pltpu.VMEM((PVOCAB,), jnp.int32),
            pltpu.VMEM((RS, W), jnp.int32),
            pltpu.VMEM((RS, W), jnp.int32),
            pltpu.VMEM((RS, W), jnp.int32),
            pltpu.VMEM((RS, W), jnp.int32),
            pltpu.VMEM((RS, W), jnp.float32),
            pltpu.VMEM((RS, W), jnp.float32),
            pltpu.SemaphoreType.DMA,
            pltpu.SemaphoreType.DMA,
            pltpu.SemaphoreType.DMA,
            pltpu.SemaphoreType.DMA,
            pltpu.SemaphoreType.DMA,
            pltpu.SemaphoreType.DMA,
        ],
    )
    return run(input_seq, packed)


# speedup vs baseline: 2.2338x; 2.2338x over previous
import functools

import jax
import jax.numpy as jnp
from jax import lax
from jax.experimental import pallas as pl
from jax.experimental.pallas import tpu as pltpu
from jax.experimental.pallas import tpu_sc as plsc

SEQ = 200
BATCH = 16384
VOCAB = 100000
LANES = 16
NC = 2
NS = 16
NW = NC * NS
CPW = BATCH // NW
W = 256
NBLOCK = CPW // W
NG = W // LANES
RS = 8
NRC = SEQ // RS
NCHUNK = NBLOCK * NRC
NIB = 8
NOB = 2


def _sc_body(seq_hbm, delta_hbm, out_hbm,
             table_sp, table_v, idx0, idx1, idx2, idx3, idx4, idx5, idx6, idx7,
             outb0, outb1,
             in_sem0, in_sem1, in_sem2, in_sem3, in_sem4, in_sem5, in_sem6,
             in_sem7, out_sem0, out_sem1):
    sid = lax.axis_index("s")
    wid = sid * NC + lax.axis_index("c")
    base = wid * CPW
    idx_refs = (idx0, idx1, idx2, idx3, idx4, idx5, idx6, idx7)
    in_sems = (in_sem0, in_sem1, in_sem2, in_sem3, in_sem4, in_sem5, in_sem6, in_sem7)
    out_refs = (outb0, outb1)
    out_sems = (out_sem0, out_sem1)

    def hbm_slice(c):
        blk = c // NRC
        rc = c - blk * NRC
        r0 = rc * RS
        c0 = base + blk * W
        return pl.ds(r0, RS), pl.ds(c0, W)

    def in_copy(c, b):
        rs, cs = hbm_slice(c)
        return pltpu.make_async_copy(seq_hbm.at[rs, cs], idx_refs[b], in_sems[b])

    def out_copy(c, b):
        rs, cs = hbm_slice(c)
        return pltpu.make_async_copy(out_refs[b], out_hbm.at[rs, cs], out_sems[b])

    for b in range(NIB):
        in_copy(b, b).start()

    @pl.when(sid == 0)
    def _():
        pltpu.sync_copy(delta_hbm, table_sp)

    plsc.subcore_barrier()
    pltpu.sync_copy(table_sp, table_v)

    def chunk_step(c, ib, ob, acc):
        in_copy(c, ib).wait()

        @pl.when(c >= NOB)
        def _():
            out_copy(c - NOB, ob).wait()

        rc = c - (c // NRC) * NRC
        reset = rc == 0
        zero = jnp.zeros((LANES,), jnp.float32)
        acc = tuple(jnp.where(reset, zero, a) for a in acc)

        idx_ref = idx_refs[ib]
        out_ref = out_refs[ob]

        def row(l, acc):
            idxs = [idx_ref[l, pl.ds(g * LANES, LANES)] for g in range(NG)]
            vals = [plsc.load_gather(table_v, [idxs[g]]) for g in range(NG)]
            new = tuple(acc[g] + vals[g] for g in range(NG))
            for g in range(NG):
                out_ref[l, pl.ds(g * LANES, LANES)] = new[g]
            return new

        acc = lax.fori_loop(0, RS, row, acc)
        out_copy(c, ob).start()

        @pl.when(c + NIB < NCHUNK)
        def _():
            in_copy(c + NIB, ib).start()

        return acc

    def quad(i, acc):
        for b in range(NIB):
            acc = chunk_step(NIB * i + b, b, b % NOB, acc)
        return acc

    zero = tuple(jnp.zeros((LANES,), jnp.float32) for _ in range(NG))
    acc = lax.fori_loop(0, NCHUNK // NIB, quad, zero)

    for c in range(NIB * (NCHUNK // NIB), NCHUNK):
        acc = chunk_step(c, c % NIB, c % NOB, acc)

    out_copy(NCHUNK - 2, (NCHUNK - 2) % NOB).wait()
    out_copy(NCHUNK - 1, (NCHUNK - 1) % NOB).wait()


def kernel(input_seq, delta):
    mesh = plsc.VectorSubcoreMesh(core_axis_name="c", subcore_axis_name="s")
    run = pl.kernel(
        _sc_body,
        mesh=mesh,
        compiler_params=pltpu.CompilerParams(needs_layout_passes=False),
        out_type=jax.ShapeDtypeStruct((SEQ, BATCH), jnp.float32),
        scratch_types=[
            pltpu.VMEM_SHARED((VOCAB,), jnp.float32),
            pltpu.VMEM((VOCAB,), jnp.float32),
            pltpu.VMEM((RS, W), jnp.int32),
            pltpu.VMEM((RS, W), jnp.int32),
            pltpu.VMEM((RS, W), jnp.int32),
            pltpu.VMEM((RS, W), jnp.int32),
            pltpu.VMEM((RS, W), jnp.int32),
            pltpu.VMEM((RS, W), jnp.int32),
            pltpu.VMEM((RS, W), jnp.int32),
            pltpu.VMEM((RS, W), jnp.int32),
            pltpu.VMEM((RS, W), jnp.float32),
            pltpu.VMEM((RS, W), jnp.float32),
            pltpu.SemaphoreType.DMA,
            pltpu.SemaphoreType.DMA,
            pltpu.SemaphoreType.DMA,
            pltpu.SemaphoreType.DMA,
            pltpu.SemaphoreType.DMA,
            pltpu.SemaphoreType.DMA,
            pltpu.SemaphoreType.DMA,
            pltpu.SemaphoreType.DMA,
            pltpu.SemaphoreType.DMA,
            pltpu.SemaphoreType.DMA,
        ],
    )
    return run(input_seq, delta)
